# baseline (speedup 1.0000x reference)
import functools

import jax
import jax.numpy as jnp
from jax import lax
from jax.experimental import pallas as pl
from jax.experimental.pallas import tpu as pltpu
from jax.experimental.pallas import tpu_sc as plsc

_B = 16384
_D = 128
_NCLS = 1000
_NPAD = 1024
_NW = 32
_CPW = _NPAD // _NW
_MOM = 0.999
_CH = 256
_LCH = 2048
_HALF = _B // 2
_SLICE = _HALF // 16
_RCAP = _HALF + _CH + 16
_DUMMY = _CPW << 14
_TEMP = 0.1
_BASE_TEMP = 0.1

_mesh = plsc.VectorSubcoreMesh(core_axis_name="c", subcore_axis_name="s")


def _shuf(x, idx):
    dn = lax.GatherDimensionNumbers(
        offset_dims=(), collapsed_slice_dims=(0,), start_index_map=(0,))
    return lax.gather(x, idx.reshape(16, 1), dn, (1,),
                      mode=lax.GatherScatterMode.PROMISE_IN_BOUNDS)


@functools.partial(
    pl.kernel,
    out_type=jax.ShapeDtypeStruct((_NPAD, _D), jnp.float32),
    mesh=_mesh,
    compiler_params=pltpu.CompilerParams(needs_layout_passes=False),
    scratch_types=[
        pltpu.VMEM((_LCH,), jnp.int32),
        pltpu.VMEM((_LCH,), jnp.int32),
        pltpu.VMEM((2 * _RCAP,), jnp.int32),
        pltpu.VMEM((_CH,), jnp.int32),
        pltpu.VMEM((_CH, _D), jnp.float32),
        pltpu.VMEM((_CPW + 1, _D), jnp.float32),
        pltpu.VMEM_SHARED((_HALF, _D), jnp.float32),
        pltpu.SemaphoreType.DMA,
        pltpu.SemaphoreType.DMA,
        pltpu.SemaphoreType.DMA,
        pltpu.SemaphoreType.DMA,
    ],
)
def _sc_ema(feat_hbm, lbl_hbm, proto_hbm, out_hbm,
            lblA, lblB, q_v, qrel_v, feat_v, prot_v, sh_feat,
            semA, semB, semS, semG):
    cid = lax.axis_index("c")
    sid = lax.axis_index("s")
    wid = sid * 2 + cid
    lo = wid * _CPW

    lbufs = (lblA, lblB)
    lsems = (semA, semB)

    stg = pltpu.async_copy(
        feat_hbm.at[pl.ds(sid * _SLICE, _SLICE)],
        sh_feat.at[pl.ds(sid * _SLICE, _SLICE)], semS)
    lcp = [pltpu.async_copy(lbl_hbm.at[pl.ds(0, _LCH)], lblA, semA)]
    pltpu.sync_copy(proto_hbm.at[pl.ds(lo, _CPW)], prot_v.at[pl.ds(0, _CPW)])

    iota16 = lax.iota(jnp.int32, 16)
    nlc = _B // _LCH
    qlens = [jnp.int32(0), jnp.int32(0)]
    for ci in range(nlc):
        lcp[ci].wait()
        if ci + 1 < nlc:
            nb = lbufs[(ci + 1) % 2]
            lcp.append(pltpu.async_copy(
                lbl_hbm.at[pl.ds((ci + 1) * _LCH, _LCH)], nb,
                lsems[(ci + 1) % 2]))
        buf = lbufs[ci % 2]
        region = 0 if ci * _LCH < _HALF else 1
        rbase = region * _RCAP

        def scan_body(i, qp, ci=ci, buf=buf, rbase=rbase):
            base = i * 16
            cloc = buf[pl.ds(base, 16)] - lo
            msk = (cloc >= 0) & (cloc < _CPW)
            inc = plsc.cumsum(msk.astype(jnp.int32))
            qval = (ci * _LCH + base + iota16) | (cloc << 14)
            plsc.store_scatter(q_v, [rbase + qp + inc - 1], qval, mask=msk)
            return qp + inc[15]

        qlens[region] = lax.fori_loop(0, _LCH // 16, scan_body, qlens[region])

    dummy16 = jnp.full((16,), _DUMMY, jnp.int32)
    for k in range(_CH // 16 + 1):
        q_v[pl.ds(qlens[0] + k * 16, 16)] = dummy16
        q_v[pl.ds(_RCAP + qlens[1] + k * 16, 16)] = dummy16

    mco = jnp.float32(_MOM)
    mcn = jnp.float32(1.0 - _MOM)

    for h in (0, 1):
        hbase = h * _HALF
        if h == 0:
            stg.wait()
        else:
            pltpu.sync_copy(
                feat_hbm.at[pl.ds(hbase + sid * _SLICE, _SLICE)],
                sh_feat.at[pl.ds(sid * _SLICE, _SLICE)])
        plsc.subcore_barrier()

        rbase = h * _RCAP
        qlen = qlens[h]
        nch = (qlen + (_CH - 1)) // _CH

        def chunk_body(g, carry, rbase=rbase, qlen=qlen, hbase=hbase):
            cstart = rbase + g * _CH
            for t in range(_CH // 16):
                qq = q_v[pl.ds(cstart + t * 16, 16)] & (_B - 1)
                qrel_v[pl.ds(t * 16, 16)] = jnp.clip(qq - hbase, 0, _HALF - 1)
            pltpu.async_copy(sh_feat.at[qrel_v], feat_v, semG).wait()
            nent = jnp.minimum(qlen - g * _CH, _CH)
            ngrp = (nent + 15) // 16

            def grp_body(gi, c2, cstart=cstart):
                gq = q_v[pl.ds(cstart + gi * 16, 16)]
                cvec = gq >> 14
                jbase = gi * 16
                for i in range(16):
                    c = cvec[i]
                    u = []
                    for k in range(_D // 16):
                        pv = prot_v[c, pl.ds(k * 16, 16)]
                        fv = feat_v[jbase + i, pl.ds(k * 16, 16)]
                        u.append(pv * mco + fv * mcn)
                    sq = u[0] * u[0]
                    for k in range(1, _D // 16):
                        sq = sq + u[k] * u[k]
                    for sh in (8, 4, 2, 1):
                        sq = sq + _shuf(sq, iota16 ^ sh)
                    t5 = 0.5 * sq
                    y = 1.5 - t5
                    y = y * (1.5 - t5 * y * y)
                    for k in range(_D // 16):
                        prot_v[c, pl.ds(k * 16, 16)] = u[k] * y
                return c2

            lax.fori_loop(0, ngrp, grp_body, jnp.int32(0))
            return carry

        lax.fori_loop(0, nch, chunk_body, jnp.int32(0))
        plsc.subcore_barrier()

    pltpu.sync_copy(prot_v.at[pl.ds(0, _CPW)], out_hbm.at[pl.ds(lo, _CPW)])


def _tc_loss_body(p_ref, o_ref):
    p = p_ref[...]
    g = lax.dot_general(
        p, p, (((1,), (1,)), ((), ())),
        preferred_element_type=jnp.float32,
        precision=lax.Precision.HIGHEST,
    )
    e = jnp.exp(g * (1.0 / _TEMP))
    r = lax.broadcasted_iota(jnp.int32, (_NPAD, _NPAD), 0)
    c = lax.broadcasted_iota(jnp.int32, (_NPAD, _NPAD), 1)
    m = ((c < _NCLS) & (c != r)).astype(jnp.float32)
    srow = jnp.sum(e * m, axis=1, keepdims=True)
    mpn = jnp.log(srow * (1.0 / (_NCLS - 1)))
    rv = lax.broadcasted_iota(jnp.int32, (_NPAD, 1), 0) < _NCLS
    loss = jnp.sum(jnp.where(rv, mpn, 0.0)) * (_TEMP / _BASE_TEMP) / _NCLS
    o_ref[0, 0] = loss


_tc_loss = pl.pallas_call(
    _tc_loss_body,
    out_shape=jax.ShapeDtypeStruct((1, 1), jnp.float32),
    out_specs=pl.BlockSpec(memory_space=pltpu.SMEM),
)


def kernel(features, labels, prototypes):
    protos_pad = jnp.pad(prototypes, ((0, _NPAD - _NCLS), (0, 0)))
    updated = _sc_ema(features, labels, protos_pad)
    return _tc_loss(updated)[0, 0]
